# Initial kernel scaffold; baseline (speedup 1.0000x reference)
import jax
import jax.numpy as jnp
from jax.experimental import pallas as pl


def kernel(mask1, mask2):
    raise NotImplementedError("write your pallas kernel here")


# speedup vs baseline: 3.2902x; 3.2902x over previous
import functools

import jax
import jax.numpy as jnp
import numpy as np
from jax import lax
from jax.experimental import pallas as pl
from jax.experimental.pallas import tpu as pltpu
from jax.experimental.pallas import tpu_sc as plsc

NCLS = 64
NSAMP = 16
DOWN = 32
NB = 8
HW = 512
GB = HW // DOWN
D = GB * GB


def _z_consts():
    k1, k2 = jax.random.split(jax.random.key(42))
    zex, zne = [], []
    for k in (k1, k2):
        u = jax.random.uniform(k, (NB, NCLS), minval=1e-20, maxval=1.0)
        g = -jnp.log(-jnp.log(u))
        zex.append(jnp.log(jnp.float32(1.0) + 1e-11) + g)
        zne.append(jnp.log(jnp.float32(0.0) + 1e-11) + g)
    return jnp.stack(zex), jnp.stack(zne)


_INV_TABLE = np.ones((272,), np.float32)
_INV_TABLE[1:257] = (np.float32(1.0) / np.arange(1, 257)).astype(np.float32)


def _body(m1_hbm, m2_hbm, zex_hbm, zne_hbm, inv_hbm,
          sm_hbm, ids_hbm,
          pix, hist, area16, areaout, winners, winhi, areanb,
          zexv, znev, cls2slot, clsinv, idxbuf, outbuf, areamg, invtab,
          shared):
    c = lax.axis_index("c")
    s = lax.axis_index("s")
    b = s // 2
    h = s % 2
    iota = lax.iota(jnp.int32, 16)
    iota64 = iota * 64
    ones_i = jnp.ones((16,), jnp.int32)
    zero16i = jnp.zeros((16,), jnp.int32)
    zero16f = jnp.zeros((16,), jnp.float32)

    def _zero(k, _):
        hist[pl.ds(k * 16, 16)] = zero16i
        area16[pl.ds(k * 16, 16)] = zero16i
        return 0
    lax.fori_loop(0, 64, _zero, 0)

    def _zero_out(q, _):
        outbuf[pl.ds(q * 16, 16)] = zero16f
        return 0
    lax.fori_loop(0, 256, _zero_out, 0)

    pltpu.sync_copy(zex_hbm.at[c, b], zexv)
    pltpu.sync_copy(zne_hbm.at[c, b], znev)

    bh0 = h * 8

    iota32 = iota * 32
    RBW = 32 * HW
    cbase = c * RBW

    def rowblock(rb, _):
        row0 = (bh0 + rb) * 32
        pltpu.sync_copy(m1_hbm.at[b, pl.ds(row0 * HW, RBW)], pix.at[pl.ds(0, RBW)])
        pltpu.sync_copy(m2_hbm.at[b, pl.ds(row0 * HW, RBW)], pix.at[pl.ds(RBW, RBW)])

        def pixloop(i, _):
            r = i // 32
            cc = i % 32
            idxv = iota32 + (cbase + r * HW + cc)
            pv = plsc.load_gather(pix, [idxv])
            plsc.addupdate_scatter(hist, [iota64 + pv], ones_i)
            return 0
        lax.fori_loop(0, 1024, pixloop, 0)

        def clsloop(cls, carry):
            cm, ca = carry
            hv = plsc.load_gather(hist, [iota64 + cls])
            plsc.store_scatter(hist, [iota64 + cls], zero16i)
            take = hv > cm
            cm = jnp.where(take, hv, cm)
            ca = jnp.where(take, cls, ca)
            return (cm, ca)
        _, ca = lax.fori_loop(
            0, 64, clsloop,
            (jnp.full((16,), -1, jnp.int32), jnp.zeros((16,), jnp.int32)))

        winners[pl.ds(rb * 16, 16)] = ca
        plsc.addupdate_scatter(area16, [iota64 + ca], ones_i)
        return 0
    lax.fori_loop(0, GB // 2, rowblock, 0)

    area_chunks = []
    for k in range(4):
        def red(l, acc, _k=k):
            return acc + area16[pl.ds(l * 64 + _k * 16, 16)]
        acc = lax.fori_loop(0, 16, red, zero16i)
        areaout[pl.ds(k * 16, 16)] = acc
        area_chunks.append(acc)

    pltpu.sync_copy(winners, shared.at[pl.ds(s * 192, 128)])
    pltpu.sync_copy(areaout, shared.at[pl.ds(s * 192 + 128, 64)])
    plsc.subcore_barrier()

    @pl.when(h == 0)
    def _stage2():
        pltpu.sync_copy(shared.at[pl.ds((s + 1) * 192, 128)], winhi)
        pltpu.sync_copy(shared.at[pl.ds((s + 1) * 192 + 128, 64)], areanb)

        pltpu.sync_copy(inv_hbm, invtab)

        zs = []
        for k in range(4):
            am = area_chunks[k] + areanb[pl.ds(k * 16, 16)]
            areamg[pl.ds(k * 16, 16)] = am
            ex = am > 0
            zk = jnp.where(ex, zexv[pl.ds(k * 16, 16)], znev[pl.ds(k * 16, 16)])
            zs.append(zk)

        NEG = jnp.float32(-3.0e38)
        BIGI = jnp.int32(1 << 20)
        idxsel = zero16i
        for t in range(16):
            gm = jnp.maximum(jnp.maximum(jnp.max(zs[0]), jnp.max(zs[1])),
                             jnp.maximum(jnp.max(zs[2]), jnp.max(zs[3])))
            cands = [jnp.min(jnp.where(zs[k] == gm, iota + 16 * k, BIGI))
                     for k in range(4)]
            sel = jnp.minimum(jnp.minimum(cands[0], cands[1]),
                              jnp.minimum(cands[2], cands[3]))
            idxsel = jnp.where(iota == t, sel, idxsel)
            zs = [jnp.where(iota + 16 * k == sel, NEG, zs[k]) for k in range(4)]

        asel = plsc.load_gather(areamg, [idxsel])
        invsel = plsc.load_gather(invtab, [jnp.maximum(asel, 1)])

        neg1 = jnp.full((16,), -1, jnp.int32)
        for k in range(4):
            cls2slot[pl.ds(k * 16, 16)] = neg1
            clsinv[pl.ds(k * 16, 16)] = zero16f
        plsc.store_scatter(cls2slot, [idxsel], iota)
        plsc.store_scatter(clsinv, [idxsel], invsel)

        for j in range(16):
            if j < 8:
                wv = winners[pl.ds(j * 16, 16)]
            else:
                wv = winhi[pl.ds((j - 8) * 16, 16)]
            slot = plsc.load_gather(cls2slot, [wv])
            val = plsc.load_gather(clsinv, [wv])
            msk = slot >= 0
            slot2 = jnp.where(msk, slot, 0)
            plsc.store_scatter(outbuf, [slot2 * D + (iota + j * 16)], val,
                               mask=msk)

        idxbuf[...] = idxsel
        pltpu.sync_copy(outbuf, sm_hbm.at[c, b])
        pltpu.sync_copy(idxbuf, ids_hbm.at[c, b])


@jax.jit
def _run(mask1, mask2):
    zex, zne = _z_consts()
    mesh = plsc.VectorSubcoreMesh(core_axis_name="c", subcore_axis_name="s",
                                  num_cores=2, num_subcores=16)
    f = pl.kernel(
        _body,
        out_type=(
            jax.ShapeDtypeStruct((2, NB, NSAMP * D), jnp.float32),
            jax.ShapeDtypeStruct((2, NB, NSAMP), jnp.int32),
        ),
        mesh=mesh,
        compiler_params=pltpu.CompilerParams(needs_layout_passes=False),
        scratch_types=[
            pltpu.VMEM((2 * 32 * HW,), jnp.int32),
            pltpu.VMEM((1024,), jnp.int32),
            pltpu.VMEM((1024,), jnp.int32),
            pltpu.VMEM((64,), jnp.int32),
            pltpu.VMEM((128,), jnp.int32),
            pltpu.VMEM((128,), jnp.int32),
            pltpu.VMEM((64,), jnp.int32),
            pltpu.VMEM((64,), jnp.float32),
            pltpu.VMEM((64,), jnp.float32),
            pltpu.VMEM((64,), jnp.int32),
            pltpu.VMEM((64,), jnp.float32),
            pltpu.VMEM((NSAMP,), jnp.int32),
            pltpu.VMEM((NSAMP * D,), jnp.float32),
            pltpu.VMEM((64,), jnp.int32),
            pltpu.VMEM((272,), jnp.float32),
            pltpu.VMEM_SHARED((16 * 192,), jnp.int32),
        ],
    )
    m1 = mask1.reshape(NB, HW * HW)
    m2 = mask2.reshape(NB, HW * HW)
    invt = jnp.asarray(_INV_TABLE)
    sm, ids = f(m1, m2, zex, zne, invt)
    sm = sm.reshape(2, NB, NSAMP, D)
    return (sm[0], sm[1], ids[0], ids[1])


def kernel(mask1, mask2):
    sm1, sm2, id1, id2 = _run(mask1, mask2)
    return sm1, sm2, id1, id2


# speedup vs baseline: 5.7140x; 1.7367x over previous
import functools

import jax
import jax.numpy as jnp
import numpy as np
from jax import lax
from jax.experimental import pallas as pl
from jax.experimental.pallas import tpu as pltpu
from jax.experimental.pallas import tpu_sc as plsc

NCLS = 64
NSAMP = 16
DOWN = 32
NB = 8
HW = 512
GB = HW // DOWN
D = GB * GB


def _z_consts():
    k1, k2 = jax.random.split(jax.random.key(42))
    zex, zne = [], []
    for k in (k1, k2):
        u = jax.random.uniform(k, (NB, NCLS), minval=1e-20, maxval=1.0)
        g = -jnp.log(-jnp.log(u))
        zex.append(jnp.log(jnp.float32(1.0) + 1e-11) + g)
        zne.append(jnp.log(jnp.float32(0.0) + 1e-11) + g)
    return jnp.stack(zex), jnp.stack(zne)


_INV_TABLE = np.ones((272,), np.float32)
_INV_TABLE[1:257] = (np.float32(1.0) / np.arange(1, 257)).astype(np.float32)


def _body(m1_hbm, m2_hbm, zex_hbm, zne_hbm, inv_hbm,
          sm_hbm, ids_hbm,
          pixa, pixb, sem0, sem1, hist, area16, areaout, winners, winhi,
          areanb, zexv, znev, cls2slot, clsinv, idxbuf, outbuf, areamg,
          invtab, shared):
    c = lax.axis_index("c")
    s = lax.axis_index("s")
    b = s // 2
    h = s % 2
    iota = lax.iota(jnp.int32, 16)
    iota64 = iota * 64
    ones_i = jnp.ones((16,), jnp.int32)
    zero16i = jnp.zeros((16,), jnp.int32)
    zero16f = jnp.zeros((16,), jnp.float32)

    @plsc.parallel_loop(0, 64, unroll=4)
    def _zero(k):
        hist[pl.ds(k * 16, 16)] = zero16i
        area16[pl.ds(k * 16, 16)] = zero16i

    @plsc.parallel_loop(0, 256, unroll=4)
    def _zero_out(q):
        outbuf[pl.ds(q * 16, 16)] = zero16f

    pltpu.sync_copy(zex_hbm.at[c, b], zexv)
    pltpu.sync_copy(zne_hbm.at[c, b], znev)

    bh0 = h * 8

    iota32 = iota * 32
    RBW = 32 * HW
    cbase = c * RBW

    def _start(rb, buf, sem):
        row0 = (bh0 + rb) * 32
        cp1 = pltpu.async_copy(m1_hbm.at[b, pl.ds(row0 * HW, RBW)],
                               buf.at[pl.ds(0, RBW)], sem)
        cp2 = pltpu.async_copy(m2_hbm.at[b, pl.ds(row0 * HW, RBW)],
                               buf.at[pl.ds(RBW, RBW)], sem)
        return cp1, cp2

    bufs = (pixa, pixb)
    sems = (sem0, sem1)
    pending = _start(0, pixa, sem0)
    for rb in range(GB // 2):
        cur = bufs[rb % 2]
        nxt = None
        if rb + 1 < GB // 2:
            nxt = _start(rb + 1, bufs[(rb + 1) % 2], sems[(rb + 1) % 2])
        pending[0].wait()
        pending[1].wait()

        @plsc.parallel_loop(0, 1024, unroll=8)
        def pixloop(i, _cur=cur):
            base = cbase + i + (i >> 5) * (HW - 32)
            pv = plsc.load_gather(_cur, [iota32 + base])
            plsc.addupdate_scatter(hist, [pv * 16 + iota], ones_i)

        @plsc.parallel_loop(
            0, 64, carry=(jnp.full((16,), -1, jnp.int32),
                          jnp.zeros((16,), jnp.int32)))
        def clsloop(cls, carry):
            cm, ca = carry
            hv = hist[pl.ds(cls * 16, 16)]
            hist[pl.ds(cls * 16, 16)] = zero16i
            take = hv > cm
            cm = jnp.where(take, hv, cm)
            ca = jnp.where(take, cls, ca)
            return (cm, ca)
        _, ca = clsloop

        winners[pl.ds(rb * 16, 16)] = ca
        plsc.addupdate_scatter(area16, [iota64 + ca], ones_i)
        pending = nxt

    area_chunks = []
    for k in range(4):
        def red(l, acc, _k=k):
            return acc + area16[pl.ds(l * 64 + _k * 16, 16)]
        acc = lax.fori_loop(0, 16, red, zero16i)
        areaout[pl.ds(k * 16, 16)] = acc
        area_chunks.append(acc)

    pltpu.sync_copy(winners, shared.at[pl.ds(s * 192, 128)])
    pltpu.sync_copy(areaout, shared.at[pl.ds(s * 192 + 128, 64)])
    plsc.subcore_barrier()

    @pl.when(h == 0)
    def _stage2():
        pltpu.sync_copy(shared.at[pl.ds((s + 1) * 192, 128)], winhi)
        pltpu.sync_copy(shared.at[pl.ds((s + 1) * 192 + 128, 64)], areanb)

        pltpu.sync_copy(inv_hbm, invtab)

        zs = []
        for k in range(4):
            am = area_chunks[k] + areanb[pl.ds(k * 16, 16)]
            areamg[pl.ds(k * 16, 16)] = am
            ex = am > 0
            zk = jnp.where(ex, zexv[pl.ds(k * 16, 16)], znev[pl.ds(k * 16, 16)])
            zs.append(zk)

        NEG = jnp.float32(-3.0e38)
        BIGI = jnp.int32(1 << 20)
        idxsel = zero16i
        for t in range(16):
            gm = jnp.maximum(jnp.maximum(jnp.max(zs[0]), jnp.max(zs[1])),
                             jnp.maximum(jnp.max(zs[2]), jnp.max(zs[3])))
            cands = [jnp.min(jnp.where(zs[k] == gm, iota + 16 * k, BIGI))
                     for k in range(4)]
            sel = jnp.minimum(jnp.minimum(cands[0], cands[1]),
                              jnp.minimum(cands[2], cands[3]))
            idxsel = jnp.where(iota == t, sel, idxsel)
            zs = [jnp.where(iota + 16 * k == sel, NEG, zs[k]) for k in range(4)]

        asel = plsc.load_gather(areamg, [idxsel])
        invsel = plsc.load_gather(invtab, [jnp.maximum(asel, 1)])

        neg1 = jnp.full((16,), -1, jnp.int32)
        for k in range(4):
            cls2slot[pl.ds(k * 16, 16)] = neg1
            clsinv[pl.ds(k * 16, 16)] = zero16f
        plsc.store_scatter(cls2slot, [idxsel], iota)
        plsc.store_scatter(clsinv, [idxsel], invsel)

        for j in range(16):
            if j < 8:
                wv = winners[pl.ds(j * 16, 16)]
            else:
                wv = winhi[pl.ds((j - 8) * 16, 16)]
            slot = plsc.load_gather(cls2slot, [wv])
            val = plsc.load_gather(clsinv, [wv])
            msk = slot >= 0
            slot2 = jnp.where(msk, slot, 0)
            plsc.store_scatter(outbuf, [slot2 * D + (iota + j * 16)], val,
                               mask=msk)

        idxbuf[...] = idxsel
        pltpu.sync_copy(outbuf, sm_hbm.at[c, b])
        pltpu.sync_copy(idxbuf, ids_hbm.at[c, b])


@jax.jit
def _run(mask1, mask2):
    zex, zne = _z_consts()
    mesh = plsc.VectorSubcoreMesh(core_axis_name="c", subcore_axis_name="s",
                                  num_cores=2, num_subcores=16)
    f = pl.kernel(
        _body,
        out_type=(
            jax.ShapeDtypeStruct((2, NB, NSAMP * D), jnp.float32),
            jax.ShapeDtypeStruct((2, NB, NSAMP), jnp.int32),
        ),
        mesh=mesh,
        compiler_params=pltpu.CompilerParams(needs_layout_passes=False),
        scratch_types=[
            pltpu.VMEM((2 * 32 * HW,), jnp.int32),
            pltpu.VMEM((2 * 32 * HW,), jnp.int32),
            pltpu.SemaphoreType.DMA,
            pltpu.SemaphoreType.DMA,
            pltpu.VMEM((1024,), jnp.int32),
            pltpu.VMEM((1024,), jnp.int32),
            pltpu.VMEM((64,), jnp.int32),
            pltpu.VMEM((128,), jnp.int32),
            pltpu.VMEM((128,), jnp.int32),
            pltpu.VMEM((64,), jnp.int32),
            pltpu.VMEM((64,), jnp.float32),
            pltpu.VMEM((64,), jnp.float32),
            pltpu.VMEM((64,), jnp.int32),
            pltpu.VMEM((64,), jnp.float32),
            pltpu.VMEM((NSAMP,), jnp.int32),
            pltpu.VMEM((NSAMP * D,), jnp.float32),
            pltpu.VMEM((64,), jnp.int32),
            pltpu.VMEM((272,), jnp.float32),
            pltpu.VMEM_SHARED((16 * 192,), jnp.int32),
        ],
    )
    m1 = mask1.reshape(NB, HW * HW)
    m2 = mask2.reshape(NB, HW * HW)
    invt = jnp.asarray(_INV_TABLE)
    sm, ids = f(m1, m2, zex, zne, invt)
    sm = sm.reshape(2, NB, NSAMP, D)
    return (sm[0], sm[1], ids[0], ids[1])


def kernel(mask1, mask2):
    sm1, sm2, id1, id2 = _run(mask1, mask2)
    return sm1, sm2, id1, id2


# speedup vs baseline: 5.7221x; 1.0014x over previous
import functools

import jax
import jax.numpy as jnp
import numpy as np
from jax import lax
from jax.experimental import pallas as pl
from jax.experimental.pallas import tpu as pltpu
from jax.experimental.pallas import tpu_sc as plsc

NCLS = 64
NSAMP = 16
DOWN = 32
NB = 8
HW = 512
GB = HW // DOWN
D = GB * GB


def _z_consts():
    k1, k2 = jax.random.split(jax.random.key(42))
    zex, zne = [], []
    for k in (k1, k2):
        u = jax.random.uniform(k, (NB, NCLS), minval=1e-20, maxval=1.0)
        g = -jnp.log(-jnp.log(u))
        zex.append(jnp.log(jnp.float32(1.0) + 1e-11) + g)
        zne.append(jnp.log(jnp.float32(0.0) + 1e-11) + g)
    return jnp.stack(zex), jnp.stack(zne)


_INV_TABLE = np.ones((272,), np.float32)
_INV_TABLE[1:257] = (np.float32(1.0) / np.arange(1, 257)).astype(np.float32)


def _body(m1_hbm, m2_hbm, zex_hbm, zne_hbm, inv_hbm,
          sm_hbm, ids_hbm,
          pixa, pixb, sem0, sem1, hist, area16, areaout, winners, winhi,
          areanb, zexv, znev, cls2slot, clsinv, idxbuf, outbuf, areamg,
          invtab, shared):
    c = lax.axis_index("c")
    s = lax.axis_index("s")
    b = s // 2
    h = s % 2
    iota = lax.iota(jnp.int32, 16)
    iota64 = iota * 64
    ones_i = jnp.ones((16,), jnp.int32)
    zero16i = jnp.zeros((16,), jnp.int32)
    zero16f = jnp.zeros((16,), jnp.float32)

    @plsc.parallel_loop(0, 64, unroll=4)
    def _zero(k):
        hist[pl.ds(k * 16, 16)] = zero16i
        area16[pl.ds(k * 16, 16)] = zero16i

    @plsc.parallel_loop(0, 256, unroll=4)
    def _zero_out(q):
        outbuf[pl.ds(q * 16, 16)] = zero16f

    pltpu.sync_copy(zex_hbm.at[c, b], zexv)
    pltpu.sync_copy(zne_hbm.at[c, b], znev)

    bh0 = h * 8

    iota32 = iota * 32
    RBW = 32 * HW
    cbase = c * RBW

    def _start(rb, buf, sem):
        row0 = (bh0 + rb) * 32
        cp1 = pltpu.async_copy(m1_hbm.at[b, pl.ds(row0 * HW, RBW)],
                               buf.at[pl.ds(0, RBW)], sem)
        cp2 = pltpu.async_copy(m2_hbm.at[b, pl.ds(row0 * HW, RBW)],
                               buf.at[pl.ds(RBW, RBW)], sem)
        return cp1, cp2

    bufs = (pixa, pixb)
    sems = (sem0, sem1)
    pending = _start(0, pixa, sem0)
    for rb in range(GB // 2):
        cur = bufs[rb % 2]
        nxt = None
        if rb + 1 < GB // 2:
            nxt = _start(rb + 1, bufs[(rb + 1) % 2], sems[(rb + 1) % 2])
        with jax.named_scope("dmawait"):
            pending[0].wait()
            pending[1].wait()

        with jax.named_scope("pix"):
            @plsc.parallel_loop(0, 1024, unroll=8)
            def pixloop(i, _cur=cur):
                base = cbase + i + (i >> 5) * (HW - 32)
                pv = plsc.load_gather(_cur, [iota32 + base])
                plsc.addupdate_scatter(hist, [pv * 16 + iota], ones_i)

        with jax.named_scope("amax"):
            @plsc.parallel_loop(
                0, 64, carry=(jnp.full((16,), -1, jnp.int32),
                              jnp.zeros((16,), jnp.int32)))
            def clsloop(cls, carry):
                cm, ca = carry
                hv = hist[pl.ds(cls * 16, 16)]
                hist[pl.ds(cls * 16, 16)] = zero16i
                take = hv > cm
                cm = jnp.where(take, hv, cm)
                ca = jnp.where(take, cls, ca)
                return (cm, ca)
            _, ca = clsloop

        winners[pl.ds(rb * 16, 16)] = ca
        plsc.addupdate_scatter(area16, [iota64 + ca], ones_i)
        pending = nxt

    area_chunks = []
    for k in range(4):
        def red(l, acc, _k=k):
            return acc + area16[pl.ds(l * 64 + _k * 16, 16)]
        acc = lax.fori_loop(0, 16, red, zero16i)
        areaout[pl.ds(k * 16, 16)] = acc
        area_chunks.append(acc)

    pltpu.sync_copy(winners, shared.at[pl.ds(s * 192, 128)])
    pltpu.sync_copy(areaout, shared.at[pl.ds(s * 192 + 128, 64)])
    plsc.subcore_barrier()

    @pl.when(h == 0)
    def _stage2():
        pltpu.sync_copy(shared.at[pl.ds((s + 1) * 192, 128)], winhi)
        pltpu.sync_copy(shared.at[pl.ds((s + 1) * 192 + 128, 64)], areanb)

        pltpu.sync_copy(inv_hbm, invtab)

        zs = []
        for k in range(4):
            am = area_chunks[k] + areanb[pl.ds(k * 16, 16)]
            areamg[pl.ds(k * 16, 16)] = am
            ex = am > 0
            zk = jnp.where(ex, zexv[pl.ds(k * 16, 16)], znev[pl.ds(k * 16, 16)])
            zs.append(zk)

        NEG = jnp.float32(-3.0e38)
        BIGI = jnp.int32(1 << 20)
        idxsel = zero16i
        for t in range(16):
            gm = jnp.maximum(jnp.maximum(jnp.max(zs[0]), jnp.max(zs[1])),
                             jnp.maximum(jnp.max(zs[2]), jnp.max(zs[3])))
            cands = [jnp.min(jnp.where(zs[k] == gm, iota + 16 * k, BIGI))
                     for k in range(4)]
            sel = jnp.minimum(jnp.minimum(cands[0], cands[1]),
                              jnp.minimum(cands[2], cands[3]))
            idxsel = jnp.where(iota == t, sel, idxsel)
            zs = [jnp.where(iota + 16 * k == sel, NEG, zs[k]) for k in range(4)]

        asel = plsc.load_gather(areamg, [idxsel])
        invsel = plsc.load_gather(invtab, [jnp.maximum(asel, 1)])

        neg1 = jnp.full((16,), -1, jnp.int32)
        for k in range(4):
            cls2slot[pl.ds(k * 16, 16)] = neg1
            clsinv[pl.ds(k * 16, 16)] = zero16f
        plsc.store_scatter(cls2slot, [idxsel], iota)
        plsc.store_scatter(clsinv, [idxsel], invsel)

        for j in range(16):
            if j < 8:
                wv = winners[pl.ds(j * 16, 16)]
            else:
                wv = winhi[pl.ds((j - 8) * 16, 16)]
            slot = plsc.load_gather(cls2slot, [wv])
            val = plsc.load_gather(clsinv, [wv])
            msk = slot >= 0
            slot2 = jnp.where(msk, slot, 0)
            plsc.store_scatter(outbuf, [slot2 * D + (iota + j * 16)], val,
                               mask=msk)

        idxbuf[...] = idxsel
        pltpu.sync_copy(outbuf, sm_hbm.at[c, b])
        pltpu.sync_copy(idxbuf, ids_hbm.at[c, b])


@jax.jit
def _run(mask1, mask2):
    zex, zne = _z_consts()
    mesh = plsc.VectorSubcoreMesh(core_axis_name="c", subcore_axis_name="s",
                                  num_cores=2, num_subcores=16)
    f = pl.kernel(
        _body,
        out_type=(
            jax.ShapeDtypeStruct((2, NB, NSAMP * D), jnp.float32),
            jax.ShapeDtypeStruct((2, NB, NSAMP), jnp.int32),
        ),
        mesh=mesh,
        compiler_params=pltpu.CompilerParams(needs_layout_passes=False),
        scratch_types=[
            pltpu.VMEM((2 * 32 * HW,), jnp.int32),
            pltpu.VMEM((2 * 32 * HW,), jnp.int32),
            pltpu.SemaphoreType.DMA,
            pltpu.SemaphoreType.DMA,
            pltpu.VMEM((1024,), jnp.int32),
            pltpu.VMEM((1024,), jnp.int32),
            pltpu.VMEM((64,), jnp.int32),
            pltpu.VMEM((128,), jnp.int32),
            pltpu.VMEM((128,), jnp.int32),
            pltpu.VMEM((64,), jnp.int32),
            pltpu.VMEM((64,), jnp.float32),
            pltpu.VMEM((64,), jnp.float32),
            pltpu.VMEM((64,), jnp.int32),
            pltpu.VMEM((64,), jnp.float32),
            pltpu.VMEM((NSAMP,), jnp.int32),
            pltpu.VMEM((NSAMP * D,), jnp.float32),
            pltpu.VMEM((64,), jnp.int32),
            pltpu.VMEM((272,), jnp.float32),
            pltpu.VMEM_SHARED((16 * 192,), jnp.int32),
        ],
    )
    m1 = mask1.reshape(NB, HW * HW)
    m2 = mask2.reshape(NB, HW * HW)
    invt = jnp.asarray(_INV_TABLE)
    sm, ids = f(m1, m2, zex, zne, invt)
    sm = sm.reshape(2, NB, NSAMP, D)
    return (sm[0], sm[1], ids[0], ids[1])


def kernel(mask1, mask2):
    sm1, sm2, id1, id2 = _run(mask1, mask2)
    return sm1, sm2, id1, id2
